# baseline (speedup 1.0000x reference)
import jax
import jax.numpy as jnp
from jax.experimental import pallas as pl
from jax.experimental.pallas import tpu as pltpu


def _ffn_body(slots_ref, x_ref, sel_ref, *refs):
    j = pl.program_id(0)
    e = pl.program_id(1)
    out_ref = refs[-1]
    wrefs = refs[:-1]
    S = len(wrefs) // 3

    @pl.when((j == 0) & (e == 0))
    def _():
        out_ref[...] = jnp.zeros_like(out_ref)

    @pl.when(slots_ref[e] == e)
    def _():
        x = x_ref[...]
        dn = (((1,), (1,)), ((), ()))
        y = None
        for s in range(S):
            w1t = wrefs[s][0]
            w3t = wrefs[S + s][0]
            w2t = wrefs[2 * S + s][0]
            h1 = jax.lax.dot_general(x, w1t, dn,
                                     preferred_element_type=jnp.float32)
            h3 = jax.lax.dot_general(x, w3t, dn,
                                     preferred_element_type=jnp.float32)
            g = h1 * jax.lax.logistic(h1) * h3
            ys = jax.lax.dot_general(g, w2t, dn,
                                     preferred_element_type=jnp.float32)
            y = ys if y is None else y + ys

        sel = sel_ref[0]
        out_ref[...] += jax.lax.dot_general(
            sel, y, (((1,), (0,)), ((), ())),
            preferred_element_type=jnp.float32)


def kernel(x, expert_indices, w1, w2, w3):
    E, I, D = w1.shape
    T, K = expert_indices.shape
    TK = T * K
    IT = 2048
    J = I // IT

    idx_flat = expert_indices.astype(jnp.int32).reshape(-1)
    iota_e = jnp.arange(E, dtype=jnp.int32)
    assign = idx_flat[None, :] == iota_e[:, None]
    used = jnp.any(assign, axis=1)
    slots = jax.lax.cummax(jnp.where(used, iota_e, -1))
    first_used = jnp.argmax(used).astype(jnp.int32)
    slots = jnp.where(slots < 0, first_used, slots).astype(jnp.int32)
    onehot = (jnp.arange(TK)[:, None] // K
              == jnp.arange(T)[None, :])
    sel = (assign[:, :, None] & onehot[None]).astype(jnp.float32)

    SPL = 1
    ITS = IT // SPL

    def _w13_spec(s_off):
        return pl.BlockSpec(
            (1, ITS, D), lambda j, e, s: (s[e], j * SPL + s_off, 0))

    def _w2_spec(s_off):
        return pl.BlockSpec(
            (1, D, ITS), lambda j, e, s: (s[e], 0, j * SPL + s_off))

    in_specs = [
        pl.BlockSpec((T, D), lambda j, e, s: (0, 0)),
        pl.BlockSpec((1, TK, T), lambda j, e, s: (e, 0, 0)),
    ]
    in_specs += [_w13_spec(s) for s in range(SPL)]
    in_specs += [_w13_spec(s) for s in range(SPL)]
    in_specs += [_w2_spec(s) for s in range(SPL)]

    grid_spec = pltpu.PrefetchScalarGridSpec(
        num_scalar_prefetch=1,
        grid=(J, E),
        in_specs=in_specs,
        out_specs=pl.BlockSpec((TK, D), lambda j, e, s: (0, 0)),
    )

    operands = ([slots, x, sel]
                + [w1] * SPL + [w3] * SPL + [w2] * SPL)
    out = pl.pallas_call(
        _ffn_body,
        grid_spec=grid_spec,
        out_shape=jax.ShapeDtypeStruct((TK, D), jnp.float32),
        compiler_params=pltpu.CompilerParams(
            vmem_limit_bytes=128 * 1024 * 1024),
    )(*operands)
    return out.reshape(T, K, D)


# speedup vs baseline: 1.1132x; 1.1132x over previous
import jax
import jax.numpy as jnp
from jax.experimental import pallas as pl
from jax.experimental.pallas import tpu as pltpu


def _ffn_body(slots_ref, valids_ref, x_ref, sel_ref, *refs):
    j = pl.program_id(0)
    e = pl.program_id(1)
    out_ref = refs[-1]
    wrefs = refs[:-1]
    S = len(wrefs) // 3

    @pl.when((j == 0) & (e == 0))
    def _():
        out_ref[...] = jnp.zeros_like(out_ref)

    @pl.when(valids_ref[e] == 1)
    def _():
        x = x_ref[...]
        dn = (((1,), (1,)), ((), ()))
        y = None
        for s in range(S):
            w1t = wrefs[s][0]
            w3t = wrefs[S + s][0]
            w2t = wrefs[2 * S + s][0]
            h1 = jax.lax.dot_general(x, w1t, dn,
                                     preferred_element_type=jnp.float32)
            h3 = jax.lax.dot_general(x, w3t, dn,
                                     preferred_element_type=jnp.float32)
            g = h1 * jax.lax.logistic(h1) * h3
            ys = jax.lax.dot_general(g, w2t, dn,
                                     preferred_element_type=jnp.float32)
            y = ys if y is None else y + ys

        sel = sel_ref[0]
        out_ref[...] += jax.lax.dot_general(
            sel, y, (((1,), (0,)), ((), ())),
            preferred_element_type=jnp.float32)


def kernel(x, expert_indices, w1, w2, w3):
    E, I, D = w1.shape
    T, K = expert_indices.shape
    TK = T * K
    IT = 2048
    J = I // IT

    idx_flat = expert_indices.astype(jnp.int32).reshape(-1)
    flat = jnp.sort(idx_flat)
    is_new = jnp.concatenate(
        [jnp.ones((1,), jnp.bool_), flat[1:] != flat[:-1]])
    pos = jnp.cumsum(is_new) - 1
    slots = jnp.full((E,), flat[-1], jnp.int32)
    slots = slots.at[pos].set(flat)
    valid = jnp.concatenate(
        [jnp.ones((1,), jnp.bool_), slots[1:] != slots[:-1]])
    assign = valid[:, None] & (idx_flat[None, :] == slots[:, None])
    onehot = (jnp.arange(TK)[:, None] // K
              == jnp.arange(T)[None, :])
    sel = (assign[:, :, None] & onehot[None]).astype(jnp.float32)

    SPL = 1
    ITS = IT // SPL

    def _w13_spec(s_off):
        return pl.BlockSpec(
            (1, ITS, D), lambda j, e, s, v: (s[e], j * SPL + s_off, 0))

    def _w2_spec(s_off):
        return pl.BlockSpec(
            (1, D, ITS), lambda j, e, s, v: (s[e], 0, j * SPL + s_off))

    in_specs = [
        pl.BlockSpec((T, D), lambda j, e, s, v: (0, 0)),
        pl.BlockSpec((1, TK, T), lambda j, e, s, v: (e, 0, 0)),
    ]
    in_specs += [_w13_spec(s) for s in range(SPL)]
    in_specs += [_w13_spec(s) for s in range(SPL)]
    in_specs += [_w2_spec(s) for s in range(SPL)]

    grid_spec = pltpu.PrefetchScalarGridSpec(
        num_scalar_prefetch=2,
        grid=(J, E),
        in_specs=in_specs,
        out_specs=pl.BlockSpec((TK, D), lambda j, e, s, v: (0, 0)),
    )

    operands = ([slots, valid.astype(jnp.int32), x, sel]
                + [w1] * SPL + [w3] * SPL + [w2] * SPL)
    out = pl.pallas_call(
        _ffn_body,
        grid_spec=grid_spec,
        out_shape=jax.ShapeDtypeStruct((TK, D), jnp.float32),
        compiler_params=pltpu.CompilerParams(
            vmem_limit_bytes=128 * 1024 * 1024),
    )(*operands)
    return out.reshape(T, K, D)
